# Initial kernel scaffold; baseline (speedup 1.0000x reference)
import jax
import jax.numpy as jnp
from jax.experimental import pallas as pl


def kernel(x, pos, batch, norm, edges_to_count, params):
    raise NotImplementedError("write your pallas kernel here")


# speedup vs baseline: 2.6191x; 2.6191x over previous
import functools
import math

import jax
import jax.numpy as jnp
import numpy as np
from jax import lax
from jax.experimental import pallas as pl
from jax.experimental.pallas import tpu as pltpu
from jax.experimental.pallas import tpu_sc as plsc

_N = 8192
_DIM_IN = 32
_NH = 64
_DIM_Z = 64
_DEPTH = 4
_RATIOS = [0.4] * _DEPTH
_RADII = [2.4, 5.4, 10.2, 20.2]
_K = 32
_EPS = 1e-12

_SC_WORKERS = 32
_SC_CH = 128


def _rup(x, m):
    return ((x + m - 1) // m) * m


def _fps_body(n, m, px_ref, py_ref, pz_ref, sel_ref, dist_ref):
    R = px_ref.shape[0]
    lin = (lax.broadcasted_iota(jnp.int32, (R, 128), 0) * 128
           + lax.broadcasted_iota(jnp.int32, (R, 128), 1))
    valid = lin < n
    dist_ref[...] = jnp.where(valid, jnp.inf, -jnp.inf).astype(jnp.float32)

    def zero_body(j, c):
        sel_ref[0, j] = 0
        return c
    lax.fori_loop(0, sel_ref.shape[1], zero_body, 0)

    px = px_ref[...]
    py = py_ref[...]
    pz = pz_ref[...]
    oh0 = lin == 0
    lx0 = jnp.sum(jnp.where(oh0, px, 0.0))
    ly0 = jnp.sum(jnp.where(oh0, py, 0.0))
    lz0 = jnp.sum(jnp.where(oh0, pz, 0.0))

    def body(i, carry):
        lx, ly, lz = carry
        dx = px - lx
        dy = py - ly
        dz = pz - lz
        d2 = dx * dx + dy * dy + dz * dz
        nd = jnp.minimum(dist_ref[...], jnp.where(valid, d2, -jnp.inf))
        dist_ref[...] = nd
        mx = jnp.max(nd)
        idx = jnp.min(jnp.where(nd == mx, lin, jnp.int32(2147483647)))
        sel_ref[0, i] = idx
        oh = lin == idx
        nlx = jnp.sum(jnp.where(oh, px, 0.0))
        nly = jnp.sum(jnp.where(oh, py, 0.0))
        nlz = jnp.sum(jnp.where(oh, pz, 0.0))
        return (nlx, nly, nlz)

    lax.fori_loop(1, m, body, (lx0, ly0, lz0))


def _fps(px, py, pz, n, m):
    R = px.shape[0]
    m_pad = _rup(m, 128)
    return pl.pallas_call(
        functools.partial(_fps_body, n, m),
        out_shape=jax.ShapeDtypeStruct((1, m_pad), jnp.int32),
        in_specs=[
            pl.BlockSpec((R, 128), lambda: (0, 0)),
            pl.BlockSpec((R, 128), lambda: (0, 0)),
            pl.BlockSpec((R, 128), lambda: (0, 0)),
        ],
        out_specs=pl.BlockSpec(memory_space=pltpu.SMEM),
        scratch_shapes=[pltpu.VMEM((R, 128), jnp.float32)],
    )(px, py, pz)


def _nbr_body(n, r2, K, qr_ref, px_ref, py_ref, pz_ref, cols_ref, vm_ref,
              d_ref):
    R = px_ref.shape[0]
    W = R * 128
    TQ = qr_ref.shape[0]
    qx = qr_ref[:, 0:1]
    qy = qr_ref[:, 1:2]
    qz = qr_ref[:, 2:3]
    lane = lax.broadcasted_iota(jnp.int32, (1, 128), 1)
    for c in range(R):
        pxr = px_ref[c:c + 1, :]
        pyr = py_ref[c:c + 1, :]
        pzr = pz_ref[c:c + 1, :]
        dx = qx - pxr
        dy = qy - pyr
        dz = qz - pzr
        d2 = dx * dx + dy * dy + dz * dz
        pidx = lane + (c * 128)
        ok = (d2 <= r2) & (pidx < n)
        d_ref[:, c * 128:(c + 1) * 128] = jnp.where(ok, d2, jnp.inf)

    linw = lax.broadcasted_iota(jnp.int32, (TQ, W), 1)
    cols = []
    vms = []
    for _ in range(K):
        D = d_ref[...]
        mn = jnp.min(D, axis=1, keepdims=True)
        isv = mn < jnp.inf
        idx = jnp.min(jnp.where(D == mn, linw, jnp.int32(2147483647)),
                      axis=1, keepdims=True)
        d_ref[...] = jnp.where(linw == idx, jnp.inf, D)
        cols.append(jnp.where(isv, idx, 0))
        vms.append(isv.astype(jnp.float32))
    cols_ref[...] = jnp.concatenate(cols, axis=1)
    vm_ref[...] = jnp.concatenate(vms, axis=1)


def _neighbors(qr, px, py, pz, n, r, K):
    R = px.shape[0]
    m_pad = qr.shape[0]
    TQ = 8
    r2 = np.float32(r * r)
    grid = (m_pad // TQ,)
    return pl.pallas_call(
        functools.partial(_nbr_body, n, r2, K),
        grid=grid,
        out_shape=(jax.ShapeDtypeStruct((m_pad, K), jnp.int32),
                   jax.ShapeDtypeStruct((m_pad, K), jnp.float32)),
        in_specs=[
            pl.BlockSpec((TQ, 16), lambda i: (i, 0)),
            pl.BlockSpec((R, 128), lambda i: (0, 0)),
            pl.BlockSpec((R, 128), lambda i: (0, 0)),
            pl.BlockSpec((R, 128), lambda i: (0, 0)),
        ],
        out_specs=(pl.BlockSpec((TQ, K), lambda i: (i, 0)),
                   pl.BlockSpec((TQ, K), lambda i: (i, 0))),
        scratch_shapes=[pltpu.VMEM((TQ, R * 128), jnp.float32)],
    )(qr, px, py, pz)


def _make_sc_gather(NT, D, B_pad):
    T = B_pad // (_SC_WORKERS * _SC_CH)
    mesh = plsc.VectorSubcoreMesh(core_axis_name="c", subcore_axis_name="s")

    @functools.partial(
        pl.kernel, mesh=mesh,
        out_type=jax.ShapeDtypeStruct((B_pad, D), jnp.float32),
        scratch_types=[
            pltpu.VMEM((_SC_CH,), jnp.int32),
            pltpu.VMEM((_SC_CH, D), jnp.float32),
            pltpu.SemaphoreType.DMA,
        ],
    )
    def k(table_hbm, idx_hbm, out_hbm, idx_v, rows_v, sem):
        wid = lax.axis_index("s") * 2 + lax.axis_index("c")
        base = wid * (T * _SC_CH)

        def body(t, c):
            off = base + t * _SC_CH
            pltpu.sync_copy(idx_hbm.at[pl.ds(off, _SC_CH)], idx_v)
            pltpu.async_copy(table_hbm.at[idx_v], rows_v, sem).wait()
            pltpu.sync_copy(rows_v, out_hbm.at[pl.ds(off, _SC_CH)])
            return c

        lax.fori_loop(0, T, body, 0)

    return k


def _sc_gather(table, idx, B):
    NT, D = table.shape
    B_pad = _rup(B, _SC_WORKERS * _SC_CH)
    idx_pad = jnp.concatenate(
        [idx[:B], jnp.zeros((B_pad - B,), jnp.int32)])
    return _make_sc_gather(NT, D, B_pad)(table, idx_pad)


def _linin_body(x_ref, w1_ref, b1_ref, w2_ref, b2_ref, o_ref):
    h = jnp.maximum(
        jnp.dot(x_ref[...], w1_ref[...],
                preferred_element_type=jnp.float32) + b1_ref[0:1, :], 0.0)
    h = jnp.maximum(
        jnp.dot(h, w2_ref[...],
                preferred_element_type=jnp.float32) + b2_ref[0:1, :], 0.0)
    o_ref[...] = h


def _lin_in(x, w1, b1, w2, b2):
    n = x.shape[0]
    TB = 1024
    return pl.pallas_call(
        _linin_body,
        grid=(n // TB,),
        out_shape=jax.ShapeDtypeStruct((n, _NH), jnp.float32),
        in_specs=[
            pl.BlockSpec((TB, _DIM_IN), lambda i: (i, 0)),
            pl.BlockSpec((_DIM_IN, _NH), lambda i: (0, 0)),
            pl.BlockSpec((8, _NH), lambda i: (0, 0)),
            pl.BlockSpec((_NH, _NH), lambda i: (0, 0)),
            pl.BlockSpec((8, _NH), lambda i: (0, 0)),
        ],
        out_specs=pl.BlockSpec((TB, _NH), lambda i: (i, 0)),
    )(x, w1, b1, w2, b2)


def _table_body(h_ref, p_ref, w1a_ref, o_ref):
    xw = jnp.dot(h_ref[...], w1a_ref[...], preferred_element_type=jnp.float32)
    pn = p_ref[:, 0:6]
    z = jnp.zeros((h_ref.shape[0], 128 - 68 - 6), jnp.float32)
    o_ref[...] = jnp.concatenate([xw, pn, z], axis=1)


def _build_table(h, p, w1a):
    n = h.shape[0]
    TB = 256
    return pl.pallas_call(
        _table_body,
        grid=(n // TB,),
        out_shape=jax.ShapeDtypeStruct((n, 128), jnp.float32),
        in_specs=[
            pl.BlockSpec((TB, _NH), lambda i: (i, 0)),
            pl.BlockSpec((TB, 16), lambda i: (i, 0)),
            pl.BlockSpec((_NH, 68), lambda i: (0, 0)),
        ],
        out_specs=pl.BlockSpec((TB, 128), lambda i: (i, 0)),
    )(h, p, w1a)


def _angle_cols(v1x, v1y, v1z, v2x, v2y, v2z):
    cx = v1y * v2z - v1z * v2y
    cy = v1z * v2x - v1x * v2z
    cz = v1x * v2y - v1y * v2x
    cn = jnp.sqrt(cx * cx + cy * cy + cz * cz + _EPS)
    dt = v1x * v2x + v1y * v2y + v1z * v2z
    return jnp.arctan2(cn, dt)


def _edge_body(K, qr_ref, g_ref, vm_ref, w1b_ref, b1_ref, w2_ref, b2_ref,
               w3_ref, b3_ref, o_ref):
    TQ = qr_ref.shape[0]
    E = TQ * K
    G = g_ref[...]
    xw = G[:, 0:68]
    pjx = G[:, 68:69]
    pjy = G[:, 69:70]
    pjz = G[:, 70:71]
    njx = G[:, 71:72]
    njy = G[:, 72:73]
    njz = G[:, 73:74]
    q = qr_ref[...]

    def rep(col):
        return jnp.broadcast_to(q[:, col:col + 1][:, None, :],
                                (TQ, K, 1)).reshape(E, 1)

    pix, piy, piz = rep(0), rep(1), rep(2)
    nix, niy, niz = rep(3), rep(4), rep(5)

    dx = pjx - pix
    dy = pjy - piy
    dz = pjz - piz
    dn = jnp.sqrt(dx * dx + dy * dy + dz * dz + _EPS)
    a1 = _angle_cols(nix, niy, niz, dx, dy, dz)
    a2 = _angle_cols(njx, njy, njz, dx, dy, dz)
    a3 = _angle_cols(nix, niy, niz, njx, njy, njz)
    zc = jnp.zeros((E, 4), jnp.float32)
    ppf = jnp.concatenate([dn, a1, a2, a3, zc], axis=1)

    pre = xw + jnp.dot(ppf, w1b_ref[...],
                       preferred_element_type=jnp.float32) + b1_ref[0:1, :]
    h1 = jnp.maximum(pre, 0.0)
    h2 = jnp.maximum(
        jnp.dot(h1, w2_ref[...],
                preferred_element_type=jnp.float32) + b2_ref[0:1, :], 0.0)

    vm = vm_ref[...]
    h3 = h2.reshape(TQ, K, 68) * vm[:, :, None]
    s = jnp.sum(h3, axis=1)
    cnt = jnp.maximum(jnp.sum(vm, axis=1, keepdims=True), 1.0)
    agg = s / cnt
    o_ref[...] = jnp.maximum(
        jnp.dot(agg, w3_ref[...],
                preferred_element_type=jnp.float32) + b3_ref[0:1, :], 0.0)


def _edge_mlp(qr, G, vm, w1b, b1, w2, b2, w3, b3, K):
    m_pad = qr.shape[0]
    TQ = 16
    return pl.pallas_call(
        functools.partial(_edge_body, K),
        grid=(m_pad // TQ,),
        out_shape=jax.ShapeDtypeStruct((m_pad, _NH), jnp.float32),
        in_specs=[
            pl.BlockSpec((TQ, 16), lambda i: (i, 0)),
            pl.BlockSpec((TQ * K, 128), lambda i: (i, 0)),
            pl.BlockSpec((TQ, K), lambda i: (i, 0)),
            pl.BlockSpec((8, 68), lambda i: (0, 0)),
            pl.BlockSpec((8, 68), lambda i: (0, 0)),
            pl.BlockSpec((68, 68), lambda i: (0, 0)),
            pl.BlockSpec((8, 68), lambda i: (0, 0)),
            pl.BlockSpec((68, _NH), lambda i: (0, 0)),
            pl.BlockSpec((8, _NH), lambda i: (0, 0)),
        ],
        out_specs=pl.BlockSpec((TQ, _NH), lambda i: (i, 0)),
    )(qr, G, vm, w1b, b1, w2, b2, w3, b3)


def _final_body(m, h_ref, w1_ref, b1_ref, w2_ref, b2_ref, o_ref):
    H = h_ref[...]
    rows = lax.broadcasted_iota(jnp.int32, H.shape, 0)
    Hm = jnp.where(rows < m, H, -jnp.inf)
    g = jnp.max(Hm, axis=0, keepdims=True)
    z = jnp.maximum(
        jnp.dot(g, w1_ref[...],
                preferred_element_type=jnp.float32) + b1_ref[0:1, :], 0.0)
    o_ref[...] = jnp.dot(z, w2_ref[...],
                         preferred_element_type=jnp.float32) + b2_ref[0:1, :]


def _final(h, m, w1, b1, w2, b2):
    mp = h.shape[0]
    return pl.pallas_call(
        functools.partial(_final_body, m),
        out_shape=jax.ShapeDtypeStruct((1, _DIM_Z), jnp.float32),
        in_specs=[
            pl.BlockSpec((mp, _NH), lambda: (0, 0)),
            pl.BlockSpec((_NH, _NH), lambda: (0, 0)),
            pl.BlockSpec((8, _NH), lambda: (0, 0)),
            pl.BlockSpec((_NH, _DIM_Z), lambda: (0, 0)),
            pl.BlockSpec((8, _DIM_Z), lambda: (0, 0)),
        ],
        out_specs=pl.BlockSpec((1, _DIM_Z), lambda: (0, 0)),
    )(h, w1, b1, w2, b2)


def _to_planes(p16, n):
    R = _rup(max((n + 127) // 128, 1), 8)
    out = []
    for c in range(3):
        col = p16[:n, c]
        col = jnp.concatenate([col, jnp.zeros((R * 128 - n,), jnp.float32)])
        out.append(col.reshape(R, 128))
    return out


def _bias8(b):
    return jnp.broadcast_to(b[None, :], (8, b.shape[0]))


def kernel(x, pos, batch, norm, edges_to_count, params):
    del batch

    (w1i, b1i), (w2i, b2i) = params['lin_in']
    h = _lin_in(x, w1i, _bias8(b1i), w2i, _bias8(b2i))

    p16 = jnp.concatenate(
        [pos, norm, jnp.zeros((_N, 10), jnp.float32)], axis=1)

    n = _N
    for lvl in range(_DEPTH):
        m = int(math.ceil(_RATIOS[lvl] * n))
        r = _RADII[lvl]
        p = params['sa'][lvl]
        (w1, b1), (w2, b2) = p['nn1']
        (w3, b3) = p['nn2'][0]
        w1a = w1[:_NH, :]
        w1b = jnp.concatenate(
            [w1[_NH:, :], jnp.zeros((4, 68), jnp.float32)], axis=0)

        px, py, pz = _to_planes(p16, n)

        sel = _fps(px, py, pz, n, m)[0]
        p128 = jnp.concatenate(
            [p16[:n], jnp.zeros((n, 112), jnp.float32)], axis=1)
        qrows_g = _sc_gather(p128, sel[:m], m)
        m_pad = _rup(m, 16)
        qr = jnp.concatenate(
            [qrows_g[:m, :16], jnp.zeros((m_pad - m, 16), jnp.float32)],
            axis=0)
        cols, vm = _neighbors(qr, px, py, pz, n, r, _K)

        n_pad = _rup(n, 256)
        h_pad = jnp.concatenate(
            [h[:n], jnp.zeros((n_pad - n, _NH), jnp.float32)], axis=0)
        p_pad = jnp.concatenate(
            [p16[:n], jnp.zeros((n_pad - n, 16), jnp.float32)], axis=0)
        table = _build_table(h_pad, p_pad, w1a)

        eidx = cols.reshape(m_pad * _K)
        G = _sc_gather(table, eidx, m_pad * _K)[:m_pad * _K]
        h = _edge_mlp(qr, G, vm, w1b, _bias8(b1), w2, _bias8(b2),
                      w3, _bias8(b3), _K)
        p16 = qr
        n = m

    (wo1, bo1), (wo2, bo2) = params['lin_out']
    z = _final(h, n, wo1, _bias8(bo1), wo2, _bias8(bo2))

    left = z[edges_to_count[:, 0], :]
    right = z[edges_to_count[:, 1], :]
    return jnp.sum(left * right)
